# baseline (speedup 1.0000x reference)
import jax
import jax.numpy as jnp
from jax.experimental import pallas as pl
from jax.experimental.pallas import tpu as pltpu

_TOKENS = 2048
_HIDDEN = 1024
_INTER = 4096
_TM = 1024
_NT = _TOKENS // _TM
_BI = 512
_NI = _INTER // _BI
_DIMS = (((1,), (1,)), ((), ()))


def _ffn_body(x_hbm, w1_hbm, w3_hbm, w2_hbm, y_hbm,
              xv, w1v, w3v, w2v, yv,
              x_sem, y_sem, w1_sem, w3_sem, w2_sem):
    def w_copies(j, buf):
        row = pl.ds(j * _BI, _BI)
        return (
            pltpu.make_async_copy(w1_hbm.at[row, :], w1v.at[buf], w1_sem.at[buf]),
            pltpu.make_async_copy(w3_hbm.at[row, :], w3v.at[buf], w3_sem.at[buf]),
            pltpu.make_async_copy(w2_hbm.at[:, row], w2v.at[buf], w2_sem.at[buf]),
        )

    def half(ref, t):
        return ref.at[pl.ds(t * _TM, _TM), :]

    x_copies = [
        pltpu.make_async_copy(half(x_hbm, t), half(xv, t), x_sem.at[t])
        for t in range(_NT)
    ]
    y_copies = [
        pltpu.make_async_copy(half(yv, t), half(y_hbm, t), y_sem.at[t])
        for t in range(_NT)
    ]
    w1c0, w3c0, w2c0 = w_copies(0, 0)
    x_copies[0].start()
    w1c0.start()
    x_copies[0].wait()
    w1c0.wait()
    w3c0.start()
    w2c0.start()
    x_copies[1].start()

    pending_w = (w1c0, w3c0, w2c0)
    for j in range(_NI):
        cur = j % 2
        if j + 1 < _NI:
            next_w = w_copies(j + 1, (j + 1) % 2)
            for c in next_w:
                c.start()
        w1c, w3c, w2c = pending_w
        if j + 1 < _NI:
            pending_w = next_w
        if j > 0:
            w1c.wait()
        w1b = w1v[cur]
        last_j = j == _NI - 1
        final_copies = []
        for t in range(_NT):
            if j == 0 and t == 1:
                x_copies[1].wait()
            if last_j and t == 1:
                subs = [slice(_TM + q * (_TM // 2), _TM + (q + 1) * (_TM // 2))
                        for q in range(2)]
            else:
                subs = [slice(t * _TM, (t + 1) * _TM)]
            for rows in subs:
                xt = xv[rows, :]
                h1 = jax.lax.dot_general(
                    xt, w1b, _DIMS, preferred_element_type=jnp.float32)
                if t == 0 and rows is subs[0]:
                    w3c.wait()
                    w3b = w3v[cur]
                h3 = jax.lax.dot_general(
                    xt, w3b, _DIMS, preferred_element_type=jnp.float32)
                g = jax.nn.silu(h1) * h3
                if t == 0 and rows is subs[0]:
                    w2c.wait()
                    w2b = w2v[cur]
                contrib = jax.lax.dot_general(
                    g, w2b, _DIMS, preferred_element_type=jnp.float32)
                if j == 0:
                    yv[rows, :] = contrib
                else:
                    yv[rows, :] += contrib
                if last_j and t == 1:
                    c = pltpu.make_async_copy(
                        yv.at[rows, :], y_hbm.at[rows, :],
                        y_sem.at[1 if rows is subs[0] else 2])
                    c.start()
                    final_copies.append(c)
            if last_j and t == 0:
                y_copies[0].start()

    y_copies[0].wait()
    for c in final_copies:
        c.wait()


def kernel(hidden_states, gate_w, w1, w2, w3):
    del gate_w
    return pl.pallas_call(
        _ffn_body,
        in_specs=[
            pl.BlockSpec(memory_space=pl.ANY),
            pl.BlockSpec(memory_space=pl.ANY),
            pl.BlockSpec(memory_space=pl.ANY),
            pl.BlockSpec(memory_space=pl.ANY),
        ],
        out_specs=pl.BlockSpec(memory_space=pl.ANY),
        out_shape=jax.ShapeDtypeStruct((_TOKENS, _HIDDEN), jnp.float32),
        scratch_shapes=[
            pltpu.VMEM((_TOKENS, _HIDDEN), jnp.float32),
            pltpu.VMEM((2, _BI, _HIDDEN), jnp.float32),
            pltpu.VMEM((2, _BI, _HIDDEN), jnp.float32),
            pltpu.VMEM((2, _HIDDEN, _BI), jnp.float32),
            pltpu.VMEM((_TOKENS, _HIDDEN), jnp.float32),
            pltpu.SemaphoreType.DMA((2,)),
            pltpu.SemaphoreType.DMA((3,)),
            pltpu.SemaphoreType.DMA((2,)),
            pltpu.SemaphoreType.DMA((2,)),
            pltpu.SemaphoreType.DMA((2,)),
        ],
    )(hidden_states, w1[0], w3[0], w2[0])
